# Initial kernel scaffold; baseline (speedup 1.0000x reference)
import jax
import jax.numpy as jnp
from jax.experimental import pallas as pl


def kernel(muti_int, noise, w_gate, w_noise, a_experts, prelu_w):
    raise NotImplementedError("write your pallas kernel here")


# speedup vs baseline: 3.0404x; 3.0404x over previous
import jax
import jax.numpy as jnp
from jax.experimental import pallas as pl

_D = 64
_C = 4
_E = 8
_K = 2
_NOISE_EPS = 0.01
_BN = 1024
_SQRT1_2 = 0.7071067811865476


def _moe_block_kernel(x_ref, noise_ref, wg_ref, wn_ref, a_ref, pwv_ref,
                      out_ref, imp_ref, load_ref, loss_ref):
    i = pl.program_id(0)
    nblocks = pl.num_programs(0)
    bn = x_ref.shape[0]

    x = x_ref[...]
    nz = noise_ref[...]
    clean = jnp.dot(x, wg_ref[...], preferred_element_type=jnp.float32)
    raw = jnp.dot(x, wn_ref[...], preferred_element_type=jnp.float32)
    std = jax.nn.softplus(raw) + _NOISE_EPS
    lg = clean + nz * std

    col = jax.lax.broadcasted_iota(jnp.int32, (bn, _E), 1)
    neg_inf = jnp.float32(-jnp.inf)

    i1 = jnp.argmax(lg, axis=1)[:, None]
    t1 = jnp.max(lg, axis=1, keepdims=True)
    oh1 = col == i1
    lg2 = jnp.where(oh1, neg_inf, lg)
    i2 = jnp.argmax(lg2, axis=1)[:, None]
    t2 = jnp.max(lg2, axis=1, keepdims=True)
    oh2 = col == i2
    lg3 = jnp.where(oh2, neg_inf, lg2)
    t3 = jnp.max(lg3, axis=1, keepdims=True)

    e2 = jnp.exp(t2 - t1)
    denom = 1.0 + e2
    gates = jnp.where(oh1, 1.0 / denom, 0.0) + jnp.where(oh2, e2 / denom, 0.0)

    inv_std = 1.0 / std
    pin = 0.5 * (1.0 + jax.lax.erf((clean - t3) * inv_std * _SQRT1_2))
    pout = 0.5 * (1.0 + jax.lax.erf((clean - t2) * inv_std * _SQRT1_2))
    prob = jnp.where(lg > t3, pin, pout)

    a = a_ref[...]
    pw = jnp.where(a >= 0, a, pwv_ref[...] * a)
    dj = jax.lax.broadcasted_iota(jnp.int32, (_D, _D * _C), 0)
    jj = jax.lax.broadcasted_iota(jnp.int32, (_D, _D * _C), 1)
    rep = (jj // _C == dj).astype(jnp.float32)
    pw_exp = jnp.dot(pw, rep, preferred_element_type=jnp.float32)
    gpw = jnp.dot(gates, pw_exp, preferred_element_type=jnp.float32)
    z = x * gpw
    jc = jax.lax.broadcasted_iota(jnp.int32, (_D * _C, _C), 0)
    cc = jax.lax.broadcasted_iota(jnp.int32, (_D * _C, _C), 1)
    sel = (jc % _C == cc).astype(jnp.float32)
    out_ref[...] = jnp.dot(z, sel, preferred_element_type=jnp.float32)

    @pl.when(i == 0)
    def _init():
        imp_ref[...] = jnp.zeros_like(imp_ref)
        load_ref[...] = jnp.zeros_like(load_ref)
        loss_ref[...] = jnp.zeros_like(loss_ref)

    imp_ref[...] += jnp.sum(gates, axis=0, keepdims=True)
    load_ref[...] += jnp.sum(prob, axis=0, keepdims=True)

    @pl.when(i == nblocks - 1)
    def _finish():
        def cv2(v):
            mean = jnp.sum(v) / _E
            var = jnp.sum((v - mean) ** 2) / (_E - 1)
            return var / (mean * mean + 1e-10)
        val = cv2(imp_ref[0, :]) + cv2(load_ref[0, :])
        loss_ref[...] = jnp.broadcast_to(val, (1, 1))


def kernel(muti_int, noise, w_gate, w_noise, a_experts, prelu_w):
    n = muti_int.shape[0]
    x = muti_int.reshape(n, _D * _C)
    a = a_experts.reshape(_E, _D)
    pwv = prelu_w.reshape(_E, 1)
    grid = n // _BN
    out, _, _, loss = pl.pallas_call(
        _moe_block_kernel,
        grid=(grid,),
        in_specs=[
            pl.BlockSpec((_BN, _D * _C), lambda i: (i, 0)),
            pl.BlockSpec((_BN, _E), lambda i: (i, 0)),
            pl.BlockSpec((_D * _C, _E), lambda i: (0, 0)),
            pl.BlockSpec((_D * _C, _E), lambda i: (0, 0)),
            pl.BlockSpec((_E, _D), lambda i: (0, 0)),
            pl.BlockSpec((_E, 1), lambda i: (0, 0)),
        ],
        out_specs=[
            pl.BlockSpec((_BN, _C), lambda i: (i, 0)),
            pl.BlockSpec((1, _E), lambda i: (0, 0)),
            pl.BlockSpec((1, _E), lambda i: (0, 0)),
            pl.BlockSpec((1, 1), lambda i: (0, 0)),
        ],
        out_shape=[
            jax.ShapeDtypeStruct((n, _C), jnp.float32),
            jax.ShapeDtypeStruct((1, _E), jnp.float32),
            jax.ShapeDtypeStruct((1, _E), jnp.float32),
            jax.ShapeDtypeStruct((1, 1), jnp.float32),
        ],
    )(x, noise, w_gate, w_noise, a, pwv)
    return out, loss[0, 0]


# speedup vs baseline: 3.2554x; 1.0707x over previous
import jax
import jax.numpy as jnp
from jax.experimental import pallas as pl

_D = 64
_C = 4
_E = 8
_K = 2
_NOISE_EPS = 0.01
_BN = 4096
_SQRT1_2 = 0.7071067811865476


def _moe_block_kernel(x_ref, noise_ref, wg_ref, wn_ref, a_ref, pwv_ref,
                      out_ref, imp_ref, load_ref, loss_ref):
    i = pl.program_id(0)
    nblocks = pl.num_programs(0)
    bn = x_ref.shape[0]

    x = x_ref[...]
    nz = noise_ref[...]
    clean = jnp.dot(x, wg_ref[...], preferred_element_type=jnp.float32)
    raw = jnp.dot(x, wn_ref[...], preferred_element_type=jnp.float32)
    std = jax.nn.softplus(raw) + _NOISE_EPS
    lg = clean + nz * std

    col = jax.lax.broadcasted_iota(jnp.int32, (bn, _E), 1)
    neg_inf = jnp.float32(-jnp.inf)

    i1 = jnp.argmax(lg, axis=1)[:, None]
    t1 = jnp.max(lg, axis=1, keepdims=True)
    oh1 = col == i1
    lg2 = jnp.where(oh1, neg_inf, lg)
    i2 = jnp.argmax(lg2, axis=1)[:, None]
    t2 = jnp.max(lg2, axis=1, keepdims=True)
    oh2 = col == i2
    lg3 = jnp.where(oh2, neg_inf, lg2)
    t3 = jnp.max(lg3, axis=1, keepdims=True)

    e2 = jnp.exp(t2 - t1)
    denom = 1.0 + e2
    gates = jnp.where(oh1, 1.0 / denom, 0.0) + jnp.where(oh2, e2 / denom, 0.0)

    inv_std = 1.0 / std
    pin = 0.5 * (1.0 + jax.lax.erf((clean - t3) * inv_std * _SQRT1_2))
    pout = 0.5 * (1.0 + jax.lax.erf((clean - t2) * inv_std * _SQRT1_2))
    prob = jnp.where(lg > t3, pin, pout)

    a = a_ref[...]
    pw = jnp.where(a >= 0, a, pwv_ref[...] * a)
    dj = jax.lax.broadcasted_iota(jnp.int32, (_D, _D * _C), 0)
    jj = jax.lax.broadcasted_iota(jnp.int32, (_D, _D * _C), 1)
    rep = (jj // _C == dj).astype(jnp.float32)
    pw_exp = jnp.dot(pw, rep, preferred_element_type=jnp.float32)
    gpw = jnp.dot(gates, pw_exp, preferred_element_type=jnp.float32)
    z = x * gpw
    jc = jax.lax.broadcasted_iota(jnp.int32, (_D * _C, _C), 0)
    cc = jax.lax.broadcasted_iota(jnp.int32, (_D * _C, _C), 1)
    sel = (jc % _C == cc).astype(jnp.float32)
    out_ref[...] = jnp.dot(z, sel, preferred_element_type=jnp.float32)

    @pl.when(i == 0)
    def _init():
        imp_ref[...] = jnp.zeros_like(imp_ref)
        load_ref[...] = jnp.zeros_like(load_ref)
        loss_ref[...] = jnp.zeros_like(loss_ref)

    imp_ref[...] += jnp.sum(gates, axis=0, keepdims=True)
    load_ref[...] += jnp.sum(prob, axis=0, keepdims=True)

    @pl.when(i == nblocks - 1)
    def _finish():
        def cv2(v):
            mean = jnp.sum(v) / _E
            var = jnp.sum((v - mean) ** 2) / (_E - 1)
            return var / (mean * mean + 1e-10)
        val = cv2(imp_ref[0, :]) + cv2(load_ref[0, :])
        loss_ref[...] = jnp.broadcast_to(val, (1, 1))


def kernel(muti_int, noise, w_gate, w_noise, a_experts, prelu_w):
    n = muti_int.shape[0]
    x = muti_int.reshape(n, _D * _C)
    a = a_experts.reshape(_E, _D)
    pwv = prelu_w.reshape(_E, 1)
    grid = n // _BN
    out, _, _, loss = pl.pallas_call(
        _moe_block_kernel,
        grid=(grid,),
        in_specs=[
            pl.BlockSpec((_BN, _D * _C), lambda i: (i, 0)),
            pl.BlockSpec((_BN, _E), lambda i: (i, 0)),
            pl.BlockSpec((_D * _C, _E), lambda i: (0, 0)),
            pl.BlockSpec((_D * _C, _E), lambda i: (0, 0)),
            pl.BlockSpec((_E, _D), lambda i: (0, 0)),
            pl.BlockSpec((_E, 1), lambda i: (0, 0)),
        ],
        out_specs=[
            pl.BlockSpec((_BN, _C), lambda i: (i, 0)),
            pl.BlockSpec((1, _E), lambda i: (0, 0)),
            pl.BlockSpec((1, _E), lambda i: (0, 0)),
            pl.BlockSpec((1, 1), lambda i: (0, 0)),
        ],
        out_shape=[
            jax.ShapeDtypeStruct((n, _C), jnp.float32),
            jax.ShapeDtypeStruct((1, _E), jnp.float32),
            jax.ShapeDtypeStruct((1, _E), jnp.float32),
            jax.ShapeDtypeStruct((1, 1), jnp.float32),
        ],
    )(x, noise, w_gate, w_noise, a, pwv)
    return out, loss[0, 0]
